# Initial kernel scaffold; baseline (speedup 1.0000x reference)
import jax
import jax.numpy as jnp
from jax.experimental import pallas as pl


def kernel(feats, edge_index, W, b):
    raise NotImplementedError("write your pallas kernel here")


# speedup vs baseline: 2.8330x; 2.8330x over previous
import functools

import jax
import jax.numpy as jnp
from jax import lax
from jax.experimental import pallas as pl
from jax.experimental.pallas import tpu as pltpu
from jax.experimental.pallas import tpu_sc as plsc

NC = 2
NS = 16
LANES = 16
NW = NC * NS


def _sc_degrees(sd2d, zeros_hbm, n_pad):
    R = sd2d.shape[1]
    rpt = R // NS
    mesh = plsc.VectorSubcoreMesh(core_axis_name="c", subcore_axis_name="s", num_cores=NC, num_subcores=NS)

    @functools.partial(
        pl.kernel,
        out_type=jax.ShapeDtypeStruct((2, NS, n_pad), jnp.float32),
        mesh=mesh,
        scratch_types=[
            pltpu.VMEM((rpt, 128), jnp.int32),
            pltpu.VMEM((n_pad,), jnp.float32),
        ],
        compiler_params=pltpu.CompilerParams(needs_layout_passes=False),
    )
    def deg_kernel(sd_hbm, zeros_h, out_hbm, idx_v, hist_v):
        c = lax.axis_index("c")
        s = lax.axis_index("s")
        pltpu.sync_copy(zeros_h, hist_v)
        pltpu.sync_copy(sd_hbm.at[c, pl.ds(s * rpt, rpt)], idx_v)
        ones16 = jnp.ones((16,), jnp.float32)

        def body(i, carry):
            r = i // 8
            g = i % 8
            vec = idx_v[r, pl.ds(g * 16, 16)]
            plsc.addupdate_scatter(hist_v, [vec], ones16)
            return carry

        lax.fori_loop(0, rpt * 8, body, 0)
        pltpu.sync_copy(hist_v, out_hbm.at[c, s])

    return deg_kernel(sd2d, zeros_hbm)


def _sc_gather_scatter(h_pad, sd2d, zeros_hbm, n_pad):
    D = h_pad.shape[1]
    R = sd2d.shape[1]
    rpt = R // NW
    zrows = n_pad // NS
    mesh = plsc.VectorSubcoreMesh(core_axis_name="c", subcore_axis_name="s", num_cores=NC, num_subcores=NS)

    @functools.partial(
        pl.kernel,
        out_type=jax.ShapeDtypeStruct((2, n_pad, D), jnp.float32),
        mesh=mesh,
        scratch_types=[
            pltpu.VMEM((rpt, 128), jnp.int32),
            pltpu.VMEM((rpt, 128), jnp.int32),
            pltpu.VMEM((128, D), jnp.float32),
            pltpu.VMEM_SHARED((n_pad, D), jnp.float32),
            pltpu.SemaphoreType.DMA,
        ],
    )
    def gs_kernel(h_hbm, sd_hbm, zeros_h, out_hbm, is_v, id_v, rows_v,
                  agg_sh, sem):
        c = lax.axis_index("c")
        s = lax.axis_index("s")
        w = c * NS + s
        pltpu.sync_copy(zeros_h, agg_sh.at[pl.ds(s * zrows, zrows)])
        pltpu.sync_copy(sd_hbm.at[0, pl.ds(w * rpt, rpt)], is_v)
        pltpu.sync_copy(sd_hbm.at[1, pl.ds(w * rpt, rpt)], id_v)
        plsc.subcore_barrier()

        def body(j, carry):
            pltpu.async_copy(h_hbm.at[is_v.at[j]], rows_v, sem).wait()
            pltpu.sync_copy(rows_v, agg_sh.at[id_v.at[j]], add=True)
            return carry

        lax.fori_loop(0, rpt, body, 0)
        plsc.subcore_barrier()
        pltpu.sync_copy(agg_sh.at[pl.ds(s * zrows, zrows)],
                        out_hbm.at[c, pl.ds(s * zrows, zrows)])

    return gs_kernel(h_pad, sd2d, zeros_hbm)


def _tc_normalize(feats, deg_src, blk):
    N, D = feats.shape

    def nk(f_ref, d_ref, o_ref):
        deg = jnp.sum(d_ref[...], axis=1, keepdims=True)
        scale = lax.rsqrt(jnp.maximum(deg, 1.0))
        o_ref[...] = f_ref[...] * scale

    return pl.pallas_call(
        nk,
        grid=(N // blk,),
        in_specs=[
            pl.BlockSpec((blk, D), lambda i: (i, 0)),
            pl.BlockSpec((blk, LANES), lambda i: (i, 0)),
        ],
        out_specs=pl.BlockSpec((blk, D), lambda i: (i, 0)),
        out_shape=jax.ShapeDtypeStruct((N, D), jnp.float32),
    )(feats, deg_src)


def _tc_finalize(parts, deg_dst, W, b2d, n_out, blk):
    D = parts.shape[2]
    DO = W.shape[1]

    def fk(p_ref, d_ref, w_ref, b_ref, o_ref):
        agg = p_ref[0] + p_ref[1]
        deg = jnp.sum(d_ref[...], axis=1, keepdims=True)
        scale = lax.rsqrt(jnp.maximum(deg, 1.0))
        acc = jnp.dot(agg * scale, w_ref[...],
                      preferred_element_type=jnp.float32) + b_ref[...]
        o_ref[...] = jnp.maximum(acc, 0.0)

    return pl.pallas_call(
        fk,
        grid=(n_out // blk,),
        in_specs=[
            pl.BlockSpec((2, blk, D), lambda i: (0, i, 0)),
            pl.BlockSpec((blk, LANES), lambda i: (i, 0)),
            pl.BlockSpec((D, DO), lambda i: (0, 0)),
            pl.BlockSpec((1, DO), lambda i: (0, 0)),
        ],
        out_specs=pl.BlockSpec((blk, DO), lambda i: (i, 0)),
        out_shape=jax.ShapeDtypeStruct((n_out, DO), jnp.float32),
    )(parts, deg_dst, W, b2d)


def kernel(feats, edge_index, W, b):
    N, D = feats.shape
    E = edge_index.shape[1]

    epad = pl.cdiv(E, 128 * NW * 8) * (128 * NW * 8)
    src = edge_index[0].astype(jnp.int32)
    dst = edge_index[1].astype(jnp.int32)
    sd = jnp.full((2, epad), N, dtype=jnp.int32)
    sd = sd.at[0, :E].set(src).at[1, :E].set(dst)
    sd2d = sd.reshape(2, epad // 128, 128)

    n_pad = pl.cdiv(N + 1, 64 * NS) * (64 * NS)

    zeros_deg = jnp.zeros((n_pad,), jnp.float32)
    zeros_agg = jnp.zeros((n_pad // NS, D), jnp.float32)

    degs = _sc_degrees(sd2d, zeros_deg, n_pad)
    degsT = jnp.swapaxes(degs, 1, 2)

    blk = 1000 if N % 1000 == 0 else 8
    h = _tc_normalize(feats, degsT[0, :N], blk)
    h_pad = jnp.pad(h, ((0, 16), (0, 0)))

    parts = _sc_gather_scatter(h_pad, sd2d, zeros_agg, n_pad)

    return _tc_finalize(parts, degsT[1], W, b.reshape(1, -1), N, blk)


# speedup vs baseline: 7.3458x; 2.5930x over previous
import functools

import jax
import jax.numpy as jnp
from jax import lax
from jax.experimental import pallas as pl
from jax.experimental.pallas import tpu as pltpu
from jax.experimental.pallas import tpu_sc as plsc

NC = 2
NS = 16
LANES = 16
NW = NC * NS


def _sc_degrees(sd2d, zeros_hbm, n_pad):
    R, width = sd2d.shape[1], sd2d.shape[2]
    rpt = R // NS
    gpr = width // 16
    mesh = plsc.VectorSubcoreMesh(core_axis_name="c", subcore_axis_name="s", num_cores=NC, num_subcores=NS)

    @functools.partial(
        pl.kernel,
        out_type=jax.ShapeDtypeStruct((2, NS, n_pad), jnp.float32),
        mesh=mesh,
        scratch_types=[
            pltpu.VMEM((rpt, width), jnp.int32),
            pltpu.VMEM((n_pad,), jnp.float32),
        ],
        compiler_params=pltpu.CompilerParams(needs_layout_passes=False),
    )
    def deg_kernel(sd_hbm, zeros_h, out_hbm, idx_v, hist_v):
        c = lax.axis_index("c")
        s = lax.axis_index("s")
        pltpu.sync_copy(zeros_h, hist_v)
        pltpu.sync_copy(sd_hbm.at[c, pl.ds(s * rpt, rpt)], idx_v)
        ones16 = jnp.ones((16,), jnp.float32)

        def body(i, carry):
            r = i // gpr
            g = i % gpr
            vec = idx_v[r, pl.ds(g * 16, 16)]
            plsc.addupdate_scatter(hist_v, [vec], ones16)
            return carry

        lax.fori_loop(0, rpt * gpr, body, 0)
        pltpu.sync_copy(hist_v, out_hbm.at[c, s])

    return deg_kernel(sd2d, zeros_hbm)


def _sc_gather_scatter(h_pad, sd2d, zeros_hbm, n_pad):
    D = h_pad.shape[1]
    R, width = sd2d.shape[1], sd2d.shape[2]
    rpt = R // NW
    zrows = n_pad // NS
    mesh = plsc.VectorSubcoreMesh(core_axis_name="c", subcore_axis_name="s", num_cores=NC, num_subcores=NS)

    nbuf = 2
    nchunk = 4
    rptc = rpt // nchunk
    assert rpt % (nbuf * nchunk) == 0 and rptc % 8 == 0

    @functools.partial(
        pl.kernel,
        out_type=jax.ShapeDtypeStruct((2, n_pad, D), jnp.float32),
        mesh=mesh,
        scratch_types=[
            pltpu.VMEM((rptc, width), jnp.int32),
            pltpu.VMEM((rptc, width), jnp.int32),
            pltpu.VMEM((width, D), jnp.float32),
            pltpu.VMEM((width, D), jnp.float32),
            pltpu.SemaphoreType.DMA,
            pltpu.SemaphoreType.DMA,
            pltpu.VMEM_SHARED((n_pad, D), jnp.float32),
        ],
    )
    def gs_kernel(h_hbm, sd_hbm, zeros_h, out_hbm, is_v, id_v, b0, b1,
                  s0, s1, agg_sh):
        bufs = (b0, b1)
        sems = (s0, s1)
        c = lax.axis_index("c")
        s = lax.axis_index("s")
        w = c * NS + s
        pltpu.sync_copy(zeros_h, agg_sh.at[pl.ds(s * zrows, zrows)])
        plsc.subcore_barrier()

        def outer(ch, carry):
            base = w * rpt + ch * rptc
            pltpu.sync_copy(sd_hbm.at[0, pl.ds(base, rptc)], is_v)
            pltpu.sync_copy(sd_hbm.at[1, pl.ds(base, rptc)], id_v)
            for b in range(nbuf):
                pltpu.async_copy(h_hbm.at[is_v.at[b]], bufs[b], sems[b])

            def body(g, carry2):
                j0 = g * nbuf
                for b in range(nbuf):
                    j = j0 + b
                    pltpu.make_async_copy(h_hbm.at[is_v.at[j]], bufs[b],
                                          sems[b]).wait()
                    pltpu.sync_copy(bufs[b], agg_sh.at[id_v.at[j]], add=True)

                    @pl.when(j + nbuf < rptc)
                    def _():
                        pltpu.async_copy(h_hbm.at[is_v.at[j + nbuf]],
                                         bufs[b], sems[b])
                return carry2

            lax.fori_loop(0, rptc // nbuf, body, 0)
            return carry

        lax.fori_loop(0, nchunk, outer, 0)
        plsc.subcore_barrier()
        pltpu.sync_copy(agg_sh.at[pl.ds(s * zrows, zrows)],
                        out_hbm.at[c, pl.ds(s * zrows, zrows)])

    return gs_kernel(h_pad, sd2d, zeros_hbm)


def _tc_normalize(feats, deg_src, blk):
    N, D = feats.shape

    def nk(f_ref, d_ref, o_ref):
        deg = jnp.sum(d_ref[...], axis=1, keepdims=True)
        scale = lax.rsqrt(jnp.maximum(deg, 1.0))
        o_ref[...] = f_ref[...] * scale

    return pl.pallas_call(
        nk,
        grid=(N // blk,),
        in_specs=[
            pl.BlockSpec((blk, D), lambda i: (i, 0)),
            pl.BlockSpec((blk, LANES), lambda i: (i, 0)),
        ],
        out_specs=pl.BlockSpec((blk, D), lambda i: (i, 0)),
        out_shape=jax.ShapeDtypeStruct((N, D), jnp.float32),
    )(feats, deg_src)


def _tc_finalize(parts, deg_dst, W, b2d, n_out, blk):
    D = parts.shape[2]
    DO = W.shape[1]

    def fk(p_ref, d_ref, w_ref, b_ref, o_ref):
        agg = p_ref[0] + p_ref[1]
        deg = jnp.sum(d_ref[...], axis=1, keepdims=True)
        scale = lax.rsqrt(jnp.maximum(deg, 1.0))
        acc = jnp.dot(agg * scale, w_ref[...],
                      preferred_element_type=jnp.float32) + b_ref[...]
        o_ref[...] = jnp.maximum(acc, 0.0)

    return pl.pallas_call(
        fk,
        grid=(n_out // blk,),
        in_specs=[
            pl.BlockSpec((2, blk, D), lambda i: (0, i, 0)),
            pl.BlockSpec((blk, LANES), lambda i: (i, 0)),
            pl.BlockSpec((D, DO), lambda i: (0, 0)),
            pl.BlockSpec((1, DO), lambda i: (0, 0)),
        ],
        out_specs=pl.BlockSpec((blk, DO), lambda i: (i, 0)),
        out_shape=jax.ShapeDtypeStruct((n_out, DO), jnp.float32),
    )(parts, deg_dst, W, b2d)


def kernel(feats, edge_index, W, b):
    N, D = feats.shape
    E = edge_index.shape[1]

    width = 64
    epad = pl.cdiv(E, width * NW * 8) * (width * NW * 8)
    src = edge_index[0].astype(jnp.int32)
    dst = edge_index[1].astype(jnp.int32)
    fill = N + (jnp.arange(epad, dtype=jnp.int32) % width)
    sd = jnp.broadcast_to(fill, (2, epad))
    sd = sd.at[0, :E].set(src).at[1, :E].set(dst)
    sd2d = sd.reshape(2, epad // width, width)

    n_pad = pl.cdiv(N + 1, 64 * NS) * (64 * NS)

    zeros_deg = jnp.zeros((n_pad,), jnp.float32)
    zeros_agg = jnp.zeros((n_pad // NS, D), jnp.float32)

    degs = _sc_degrees(sd2d, zeros_deg, n_pad)
    degsT = jnp.swapaxes(degs, 1, 2)

    blk = 1000 if N % 1000 == 0 else 8
    h = _tc_normalize(feats, degsT[0, :N], blk)
    h_pad = jnp.pad(h, ((0, width), (0, 0)))

    parts = _sc_gather_scatter(h_pad, sd2d, zeros_agg, n_pad)

    return _tc_finalize(parts, degsT[1], W, b.reshape(1, -1), N, blk)


# speedup vs baseline: 7.5848x; 1.0325x over previous
import functools

import jax
import jax.numpy as jnp
from jax import lax
from jax.experimental import pallas as pl
from jax.experimental.pallas import tpu as pltpu
from jax.experimental.pallas import tpu_sc as plsc

NC = 2
NS = 16
LANES = 16
NW = NC * NS


def _sc_degrees(sd2d, zeros_hbm, n_pad):
    R, width = sd2d.shape[1], sd2d.shape[2]
    rpt = R // NS
    gpr = width // 16
    mesh = plsc.VectorSubcoreMesh(core_axis_name="c", subcore_axis_name="s", num_cores=NC, num_subcores=NS)

    @functools.partial(
        pl.kernel,
        out_type=jax.ShapeDtypeStruct((2, NS, n_pad), jnp.float32),
        mesh=mesh,
        scratch_types=[
            pltpu.VMEM((rpt, width), jnp.int32),
            pltpu.VMEM((n_pad,), jnp.float32),
        ],
        compiler_params=pltpu.CompilerParams(needs_layout_passes=False),
    )
    def deg_kernel(sd_hbm, zeros_h, out_hbm, idx_v, hist_v):
        c = lax.axis_index("c")
        s = lax.axis_index("s")
        pltpu.sync_copy(zeros_h, hist_v)
        pltpu.sync_copy(sd_hbm.at[c, pl.ds(s * rpt, rpt)], idx_v)
        ones16 = jnp.ones((16,), jnp.float32)

        def body(i, carry):
            r = i // gpr
            g = i % gpr
            vec = idx_v[r, pl.ds(g * 16, 16)]
            plsc.addupdate_scatter(hist_v, [vec], ones16)
            return carry

        lax.fori_loop(0, rpt * gpr, body, 0)
        pltpu.sync_copy(hist_v, out_hbm.at[c, s])

    return deg_kernel(sd2d, zeros_hbm)


def _sc_gather_scatter(h_pad, sd2d, zeros_hbm, n_pad):
    D = h_pad.shape[1]
    R, width = sd2d.shape[1], sd2d.shape[2]
    rpt = R // NW
    zrows = n_pad // NS
    mesh = plsc.VectorSubcoreMesh(core_axis_name="c", subcore_axis_name="s", num_cores=NC, num_subcores=NS)

    nslot = 4
    nchunk = 4
    rptc = rpt // nchunk
    assert rpt % (nslot * nchunk) == 0 and rptc % 8 == 0

    @functools.partial(
        pl.kernel,
        out_type=jax.ShapeDtypeStruct((2, n_pad, D), jnp.float32),
        mesh=mesh,
        scratch_types=[
            pltpu.VMEM((rptc, width), jnp.int32),
            pltpu.VMEM((rptc, width), jnp.int32),
            pltpu.VMEM((width, D), jnp.float32),
            pltpu.VMEM((width, D), jnp.float32),
            pltpu.VMEM((width, D), jnp.float32),
            pltpu.VMEM((width, D), jnp.float32),
            pltpu.SemaphoreType.DMA,
            pltpu.SemaphoreType.DMA,
            pltpu.SemaphoreType.DMA,
            pltpu.SemaphoreType.DMA,
            pltpu.SemaphoreType.DMA,
            pltpu.SemaphoreType.DMA,
            pltpu.SemaphoreType.DMA,
            pltpu.SemaphoreType.DMA,
            pltpu.VMEM_SHARED((n_pad, D), jnp.float32),
        ],
    )
    def gs_kernel(h_hbm, sd_hbm, zeros_h, out_hbm, is_v, id_v,
                  b0, b1, b2, b3, g0, g1, g2, g3, t0, t1, t2, t3, agg_sh):
        bufs = (b0, b1, b2, b3)
        gsem = (g0, g1, g2, g3)
        ssem = (t0, t1, t2, t3)
        c = lax.axis_index("c")
        s = lax.axis_index("s")
        w = c * NS + s
        pltpu.sync_copy(zeros_h, agg_sh.at[pl.ds(s * zrows, zrows)])
        plsc.subcore_barrier()

        def outer(ch, carry):
            base = w * rpt + ch * rptc
            pltpu.sync_copy(sd_hbm.at[0, pl.ds(base, rptc)], is_v)
            pltpu.sync_copy(sd_hbm.at[1, pl.ds(base, rptc)], id_v)
            for b in range(2):
                pltpu.async_copy(h_hbm.at[is_v.at[b]], bufs[b], gsem[b])

            def body(g, carry2):
                for u in range(nslot):
                    j = g * nslot + u
                    b = u
                    nb = (u + 2) % nslot
                    pltpu.make_async_copy(h_hbm.at[is_v.at[j]], bufs[b],
                                          gsem[b]).wait()
                    pltpu.async_copy(bufs[b], agg_sh.at[id_v.at[j]],
                                     ssem[b], add=True)

                    @pl.when(j >= 2)
                    def _():
                        pltpu.make_async_copy(
                            bufs[nb], agg_sh.at[id_v.at[j]], ssem[nb]).wait()

                    @pl.when(j + 2 < rptc)
                    def _():
                        pltpu.async_copy(h_hbm.at[is_v.at[j + 2]],
                                         bufs[nb], gsem[nb])
                return carry2

            lax.fori_loop(0, rptc // nslot, body, 0)
            for j in (rptc - 2, rptc - 1):
                b = j % nslot
                pltpu.make_async_copy(bufs[b], agg_sh.at[id_v.at[j]],
                                      ssem[b]).wait()
            return carry

        lax.fori_loop(0, nchunk, outer, 0)
        plsc.subcore_barrier()
        pltpu.sync_copy(agg_sh.at[pl.ds(s * zrows, zrows)],
                        out_hbm.at[c, pl.ds(s * zrows, zrows)])

    return gs_kernel(h_pad, sd2d, zeros_hbm)


def _tc_normalize(feats, deg_src, n_pad, blk):
    N, D = feats.shape

    def nk(f_ref, d_ref, o_ref):
        deg = jnp.sum(d_ref[...], axis=1, keepdims=True)
        scale = lax.rsqrt(jnp.maximum(deg, 1.0))
        o_ref[...] = f_ref[...] * scale

    return pl.pallas_call(
        nk,
        grid=(n_pad // blk,),
        in_specs=[
            pl.BlockSpec((blk, D), lambda i: (i, 0)),
            pl.BlockSpec((blk, LANES), lambda i: (i, 0)),
        ],
        out_specs=pl.BlockSpec((blk, D), lambda i: (i, 0)),
        out_shape=jax.ShapeDtypeStruct((n_pad, D), jnp.float32),
    )(feats, deg_src)


def _tc_finalize(parts, deg_dst, W, b2d, n_out, blk):
    D = parts.shape[2]
    DO = W.shape[1]

    def fk(p_ref, d_ref, w_ref, b_ref, o_ref):
        agg = p_ref[0] + p_ref[1]
        deg = jnp.sum(d_ref[...], axis=1, keepdims=True)
        scale = lax.rsqrt(jnp.maximum(deg, 1.0))
        acc = jnp.dot(agg * scale, w_ref[...],
                      preferred_element_type=jnp.float32) + b_ref[...]
        o_ref[...] = jnp.maximum(acc, 0.0)

    return pl.pallas_call(
        fk,
        grid=(n_out // blk,),
        in_specs=[
            pl.BlockSpec((2, blk, D), lambda i: (0, i, 0)),
            pl.BlockSpec((blk, LANES), lambda i: (i, 0)),
            pl.BlockSpec((D, DO), lambda i: (0, 0)),
            pl.BlockSpec((1, DO), lambda i: (0, 0)),
        ],
        out_specs=pl.BlockSpec((blk, DO), lambda i: (i, 0)),
        out_shape=jax.ShapeDtypeStruct((n_out, DO), jnp.float32),
    )(parts, deg_dst, W, b2d)


def kernel(feats, edge_index, W, b):
    N, D = feats.shape
    E = edge_index.shape[1]

    width = 64
    epad = pl.cdiv(E, width * NW * 8) * (width * NW * 8)
    src = edge_index[0].astype(jnp.int32)
    dst = edge_index[1].astype(jnp.int32)
    fill = N + (jnp.arange(epad, dtype=jnp.int32) % width)
    sd = jnp.broadcast_to(fill, (2, epad))
    sd = sd.at[0, :E].set(src).at[1, :E].set(dst)
    sd2d = sd.reshape(2, epad // width, width)

    n_pad = pl.cdiv(N + 1, 64 * NS) * (64 * NS)

    zeros_deg = jnp.zeros((n_pad,), jnp.float32)
    zeros_agg = jnp.zeros((n_pad // NS, D), jnp.float32)

    degs = _sc_degrees(sd2d, zeros_deg, n_pad)
    degsT = jnp.swapaxes(degs, 1, 2)

    blk = 1000 if N % 1000 == 0 else 8
    h_pad = _tc_normalize(feats, degsT[0], n_pad, 512)

    parts = _sc_gather_scatter(h_pad, sd2d, zeros_agg, n_pad)

    return _tc_finalize(parts, degsT[1], W, b.reshape(1, -1), N, blk)


# speedup vs baseline: 8.1633x; 1.0763x over previous
import functools

import jax
import jax.numpy as jnp
from jax import lax
from jax.experimental import pallas as pl
from jax.experimental.pallas import tpu as pltpu
from jax.experimental.pallas import tpu_sc as plsc

NC = 2
NS = 16
LANES = 16
NW = NC * NS


def _sc_degrees(sd2d, n_pad):
    R, width = sd2d.shape[1], sd2d.shape[2]
    rpt = R // NS
    gpr = width // 16
    mesh = plsc.VectorSubcoreMesh(core_axis_name="c", subcore_axis_name="s", num_cores=NC, num_subcores=NS)

    @functools.partial(
        pl.kernel,
        out_type=jax.ShapeDtypeStruct((2, NS, n_pad), jnp.float32),
        mesh=mesh,
        scratch_types=[
            pltpu.VMEM((rpt, width), jnp.int32),
            pltpu.VMEM((n_pad,), jnp.float32),
        ],
        compiler_params=pltpu.CompilerParams(needs_layout_passes=False),
    )
    def deg_kernel(sd_hbm, out_hbm, idx_v, hist_v):
        c = lax.axis_index("c")
        s = lax.axis_index("s")
        pltpu.sync_copy(sd_hbm.at[c, pl.ds(s * rpt, rpt)], idx_v)
        z16 = jnp.zeros((16,), jnp.float32)

        def zbody(i, carry):
            hist_v[pl.ds(i * 16, 16)] = z16
            return carry

        lax.fori_loop(0, n_pad // 16, zbody, 0)
        ones16 = jnp.ones((16,), jnp.float32)

        def body(i, carry):
            r = i // gpr
            g = i % gpr
            vec = idx_v[r, pl.ds(g * 16, 16)]
            plsc.addupdate_scatter(hist_v, [vec], ones16)
            return carry

        lax.fori_loop(0, rpt * gpr, body, 0)
        pltpu.sync_copy(hist_v, out_hbm.at[c, s])

    return deg_kernel(sd2d)


def _sc_gather_scatter(h_pad, sd2d, n_pad):
    D = h_pad.shape[1]
    R, width = sd2d.shape[1], sd2d.shape[2]
    rpt = R // NW
    zrows = n_pad // NS
    mesh = plsc.VectorSubcoreMesh(core_axis_name="c", subcore_axis_name="s", num_cores=NC, num_subcores=NS)

    nslot = 4
    gdep = 3
    nchunk = 4
    rptc = rpt // nchunk
    assert rpt % (nslot * nchunk) == 0 and rptc % 8 == 0

    @functools.partial(
        pl.kernel,
        out_type=jax.ShapeDtypeStruct((2, n_pad, D), jnp.float32),
        mesh=mesh,
        scratch_types=[
            pltpu.VMEM((2, rptc, width), jnp.int32),
            pltpu.VMEM((width, D), jnp.float32),
            pltpu.VMEM((width, D), jnp.float32),
            pltpu.VMEM((width, D), jnp.float32),
            pltpu.VMEM((width, D), jnp.float32),
            pltpu.SemaphoreType.DMA,
            pltpu.SemaphoreType.DMA,
            pltpu.SemaphoreType.DMA,
            pltpu.SemaphoreType.DMA,
            pltpu.SemaphoreType.DMA,
            pltpu.SemaphoreType.DMA,
            pltpu.SemaphoreType.DMA,
            pltpu.SemaphoreType.DMA,
            pltpu.VMEM_SHARED((n_pad, D), jnp.float32),
        ],
    )
    def gs_kernel(h_hbm, sd_hbm, out_hbm, idx_v,
                  b0, b1, b2, b3, g0, g1, g2, g3,
                  t0, t1, t2, t3, agg_sh):
        bufs = (b0, b1, b2, b3)
        gsem = (g0, g1, g2, g3)
        ssem = (t0, t1, t2, t3)
        c = lax.axis_index("c")
        s = lax.axis_index("s")
        w = c * NS + s
        z16 = jnp.zeros((16,), jnp.float32)

        def zbody(i, carry):
            b0[i // (D // 16), pl.ds((i % (D // 16)) * 16, 16)] = z16
            return carry

        lax.fori_loop(0, width * (D // 16), zbody, 0)
        for k in range(zrows // width):
            pltpu.sync_copy(b0, agg_sh.at[pl.ds(s * zrows + k * width, width)])
        plsc.subcore_barrier()

        def outer(ch, carry):
            base = w * rpt + ch * rptc
            pltpu.sync_copy(sd_hbm.at[0, pl.ds(base, rptc)], idx_v.at[0])
            pltpu.sync_copy(sd_hbm.at[1, pl.ds(base, rptc)], idx_v.at[1])
            for b in range(gdep):
                pltpu.async_copy(h_hbm.at[idx_v.at[0, b]], bufs[b], gsem[b])

            def body(g, carry2):
                for u in range(nslot):
                    j = g * nslot + u
                    b = u
                    nb = (u + gdep) % nslot
                    pltpu.make_async_copy(h_hbm.at[idx_v.at[0, j]], bufs[b],
                                          gsem[b]).wait()
                    pltpu.async_copy(bufs[b], agg_sh.at[idx_v.at[1, j]],
                                     ssem[b], add=True)

                    @pl.when(j >= nslot - gdep)
                    def _():
                        pltpu.make_async_copy(
                            bufs[nb], agg_sh.at[idx_v.at[1, j]], ssem[nb]).wait()

                    @pl.when(j + gdep < rptc)
                    def _():
                        pltpu.async_copy(h_hbm.at[idx_v.at[0, j + gdep]],
                                         bufs[nb], gsem[nb])
                return carry2

            lax.fori_loop(0, rptc // nslot, body, 0)
            for j in range(rptc - (nslot - gdep), rptc):
                b = j % nslot
                pltpu.make_async_copy(bufs[b], agg_sh.at[idx_v.at[1, j]],
                                      ssem[b]).wait()
            return carry

        lax.fori_loop(0, nchunk, outer, 0)
        plsc.subcore_barrier()
        pltpu.sync_copy(agg_sh.at[pl.ds(s * zrows, zrows)],
                        out_hbm.at[c, pl.ds(s * zrows, zrows)])

    return gs_kernel(h_pad, sd2d)


def _tc_normalize(feats, deg_src, n_pad, blk):
    N, D = feats.shape

    def nk(f_ref, d_ref, o_ref):
        deg = jnp.sum(d_ref[...], axis=1, keepdims=True)
        scale = lax.rsqrt(jnp.maximum(deg, 1.0))
        o_ref[...] = f_ref[...] * scale

    return pl.pallas_call(
        nk,
        grid=(n_pad // blk,),
        in_specs=[
            pl.BlockSpec((blk, D), lambda i: (i, 0)),
            pl.BlockSpec((blk, LANES), lambda i: (i, 0)),
        ],
        out_specs=pl.BlockSpec((blk, D), lambda i: (i, 0)),
        out_shape=jax.ShapeDtypeStruct((n_pad, D), jnp.float32),
    )(feats, deg_src)


def _tc_finalize(parts, deg_dst, W, b2d, n_out, blk):
    D = parts.shape[2]
    DO = W.shape[1]

    def fk(p_ref, d_ref, w_ref, b_ref, o_ref):
        agg = p_ref[0] + p_ref[1]
        deg = jnp.sum(d_ref[...], axis=1, keepdims=True)
        scale = lax.rsqrt(jnp.maximum(deg, 1.0))
        acc = jnp.dot(agg * scale, w_ref[...],
                      preferred_element_type=jnp.float32) + b_ref[...]
        o_ref[...] = jnp.maximum(acc, 0.0)

    return pl.pallas_call(
        fk,
        grid=(n_out // blk,),
        in_specs=[
            pl.BlockSpec((2, blk, D), lambda i: (0, i, 0)),
            pl.BlockSpec((blk, LANES), lambda i: (i, 0)),
            pl.BlockSpec((D, DO), lambda i: (0, 0)),
            pl.BlockSpec((1, DO), lambda i: (0, 0)),
        ],
        out_specs=pl.BlockSpec((blk, DO), lambda i: (i, 0)),
        out_shape=jax.ShapeDtypeStruct((n_out, DO), jnp.float32),
    )(parts, deg_dst, W, b2d)


def kernel(feats, edge_index, W, b):
    N, D = feats.shape
    E = edge_index.shape[1]

    width = 64
    epad = pl.cdiv(E, width * NW * 8) * (width * NW * 8)
    src = edge_index[0].astype(jnp.int32)
    dst = edge_index[1].astype(jnp.int32)
    fill = N + (jnp.arange(epad, dtype=jnp.int32) % width)
    sd = jnp.broadcast_to(fill, (2, epad))
    sd = sd.at[0, :E].set(src).at[1, :E].set(dst)
    sd2d = sd.reshape(2, epad // width, width)

    n_pad = pl.cdiv(N + 1, 64 * NS) * (64 * NS)

    degs = _sc_degrees(sd2d, n_pad)
    degsT = jnp.swapaxes(degs, 1, 2)

    blk = 1000 if N % 1000 == 0 else 8
    h_pad = _tc_normalize(feats, degsT[0], n_pad, 512)

    parts = _sc_gather_scatter(h_pad, sd2d, n_pad)

    return _tc_finalize(parts, degsT[1], W, b.reshape(1, -1), N, blk)


# speedup vs baseline: 10.6070x; 1.2994x over previous
import functools

import jax
import jax.numpy as jnp
from jax import lax
from jax.experimental import pallas as pl
from jax.experimental.pallas import tpu as pltpu
from jax.experimental.pallas import tpu_sc as plsc

NC = 2
NS = 16
LANES = 16
NW = NC * NS


def _sc_degrees(sd2d, n_pad):
    R, width = sd2d.shape[1], sd2d.shape[2]
    rpt = R // NS
    gpr = width // 16
    mesh = plsc.VectorSubcoreMesh(core_axis_name="c", subcore_axis_name="s", num_cores=NC, num_subcores=NS)

    @functools.partial(
        pl.kernel,
        out_type=jax.ShapeDtypeStruct((2, NS, n_pad), jnp.float32),
        mesh=mesh,
        scratch_types=[
            pltpu.VMEM((rpt, width), jnp.int32),
            pltpu.VMEM((n_pad,), jnp.float32),
        ],
        compiler_params=pltpu.CompilerParams(needs_layout_passes=False),
    )
    def deg_kernel(sd_hbm, out_hbm, idx_v, hist_v):
        c = lax.axis_index("c")
        s = lax.axis_index("s")
        pltpu.sync_copy(sd_hbm.at[c, pl.ds(s * rpt, rpt)], idx_v)
        z16 = jnp.zeros((16,), jnp.float32)

        def zbody(i, carry):
            hist_v[pl.ds(i * 16, 16)] = z16
            return carry

        lax.fori_loop(0, n_pad // 16, zbody, 0)
        ones16 = jnp.ones((16,), jnp.float32)

        def body(i, carry):
            r = i // gpr
            g = i % gpr
            vec = idx_v[r, pl.ds(g * 16, 16)]
            plsc.addupdate_scatter(hist_v, [vec], ones16)
            return carry

        lax.fori_loop(0, rpt * gpr, body, 0)
        pltpu.sync_copy(hist_v, out_hbm.at[c, s])

    return deg_kernel(sd2d)


def _sc_gather_scatter(h_pad, sd2d, n_pad):
    D = h_pad.shape[1]
    R, width = sd2d.shape[1], sd2d.shape[2]
    rpt = R // NW
    zrows = n_pad // NS
    mesh = plsc.VectorSubcoreMesh(core_axis_name="c", subcore_axis_name="s", num_cores=NC, num_subcores=NS)

    nslot = 4
    gdep = 3
    nchunk = 4
    rptc = rpt // nchunk
    assert rpt % (nslot * nchunk) == 0 and rptc % 8 == 0

    @functools.partial(
        pl.kernel,
        out_type=jax.ShapeDtypeStruct((2, n_pad, D), jnp.float32),
        mesh=mesh,
        scratch_types=[
            pltpu.VMEM((2, rptc, width), jnp.int32),
            pltpu.VMEM((width, D), jnp.float32),
            pltpu.VMEM((width, D), jnp.float32),
            pltpu.VMEM((width, D), jnp.float32),
            pltpu.VMEM((width, D), jnp.float32),
            pltpu.SemaphoreType.DMA,
            pltpu.SemaphoreType.DMA,
            pltpu.SemaphoreType.DMA,
            pltpu.SemaphoreType.DMA,
            pltpu.SemaphoreType.DMA,
            pltpu.SemaphoreType.DMA,
            pltpu.SemaphoreType.DMA,
            pltpu.SemaphoreType.DMA,
            pltpu.VMEM_SHARED((n_pad, D), jnp.float32),
        ],
    )
    def gs_kernel(h_hbm, sd_hbm, out_hbm, idx_v,
                  b0, b1, b2, b3, g0, g1, g2, g3,
                  t0, t1, t2, t3, agg_sh):
        bufs = (b0, b1, b2, b3)
        gsem = (g0, g1, g2, g3)
        ssem = (t0, t1, t2, t3)
        c = lax.axis_index("c")
        s = lax.axis_index("s")
        w = c * NS + s
        z16 = jnp.zeros((16,), jnp.float32)

        def zbody(i, carry):
            b0[i // (D // 16), pl.ds((i % (D // 16)) * 16, 16)] = z16
            return carry

        lax.fori_loop(0, width * (D // 16), zbody, 0)
        for k in range(zrows // width):
            pltpu.sync_copy(b0, agg_sh.at[pl.ds(s * zrows + k * width, width)])
        plsc.subcore_barrier()

        def outer(ch, carry):
            base = w * rpt + ch * rptc
            pltpu.sync_copy(sd_hbm.at[0, pl.ds(base, rptc)], idx_v.at[0])
            pltpu.sync_copy(sd_hbm.at[1, pl.ds(base, rptc)], idx_v.at[1])
            for b in range(gdep):
                pltpu.async_copy(h_hbm.at[idx_v.at[0, b]], bufs[b], gsem[b])

            def body(g, carry2):
                for u in range(nslot):
                    j = g * nslot + u
                    b = u
                    nb = (u + gdep) % nslot
                    pltpu.make_async_copy(h_hbm.at[idx_v.at[0, j]], bufs[b],
                                          gsem[b]).wait()
                    pltpu.async_copy(bufs[b], agg_sh.at[idx_v.at[1, j]],
                                     ssem[b], add=True)

                    @pl.when(j >= nslot - gdep)
                    def _():
                        pltpu.make_async_copy(
                            bufs[nb], agg_sh.at[idx_v.at[1, j]], ssem[nb]).wait()

                    @pl.when(j + gdep < rptc)
                    def _():
                        pltpu.async_copy(h_hbm.at[idx_v.at[0, j + gdep]],
                                         bufs[nb], gsem[nb])
                return carry2

            lax.fori_loop(0, rptc // nslot, body, 0)
            for j in range(rptc - (nslot - gdep), rptc):
                b = j % nslot
                pltpu.make_async_copy(bufs[b], agg_sh.at[idx_v.at[1, j]],
                                      ssem[b]).wait()
            return carry

        lax.fori_loop(0, nchunk, outer, 0)
        plsc.subcore_barrier()
        pltpu.sync_copy(agg_sh.at[pl.ds(s * zrows, zrows)],
                        out_hbm.at[c, pl.ds(s * zrows, zrows)])

    return gs_kernel(h_pad, sd2d)


def _tc_normalize(feats, deg_src, n_pad, blk):
    N, D = feats.shape

    def nk(f_ref, d_ref, o_ref):
        deg = jnp.sum(d_ref[...], axis=0)
        scale = lax.rsqrt(jnp.maximum(deg, 1.0)).reshape(blk, 1)
        o_ref[...] = f_ref[...] * scale

    return pl.pallas_call(
        nk,
        grid=(n_pad // blk,),
        in_specs=[
            pl.BlockSpec((blk, D), lambda i: (i, 0)),
            pl.BlockSpec((NS, blk), lambda i: (0, i)),
        ],
        out_specs=pl.BlockSpec((blk, D), lambda i: (i, 0)),
        out_shape=jax.ShapeDtypeStruct((n_pad, D), jnp.float32),
    )(feats, deg_src)


def _tc_finalize(parts, deg_dst, W, b2d, n_out, blk):
    D = parts.shape[2]
    DO = W.shape[1]

    def fk(p_ref, d_ref, w_ref, b_ref, o_ref):
        agg = p_ref[0] + p_ref[1]
        deg = jnp.sum(d_ref[...], axis=0)
        scale = lax.rsqrt(jnp.maximum(deg, 1.0)).reshape(d_ref.shape[1], 1)
        acc = jnp.dot(agg * scale, w_ref[...],
                      preferred_element_type=jnp.float32) + b_ref[...]
        o_ref[...] = jnp.maximum(acc, 0.0)

    return pl.pallas_call(
        fk,
        grid=(pl.cdiv(n_out, blk),),
        in_specs=[
            pl.BlockSpec((2, blk, D), lambda i: (0, i, 0)),
            pl.BlockSpec((NS, blk), lambda i: (0, i)),
            pl.BlockSpec((D, DO), lambda i: (0, 0)),
            pl.BlockSpec((1, DO), lambda i: (0, 0)),
        ],
        out_specs=pl.BlockSpec((blk, DO), lambda i: (i, 0)),
        out_shape=jax.ShapeDtypeStruct((n_out, DO), jnp.float32),
    )(parts, deg_dst, W, b2d)


def kernel(feats, edge_index, W, b):
    N, D = feats.shape
    E = edge_index.shape[1]

    width = 64
    epad = pl.cdiv(E, width * NW * 8) * (width * NW * 8)
    fill = N + (jnp.arange(epad - E, dtype=jnp.int32) % width)
    tail = jnp.broadcast_to(fill, (2, epad - E))
    sd = jnp.concatenate([edge_index.astype(jnp.int32), tail], axis=1)
    sd2d = sd.reshape(2, epad // width, width)

    n_pad = pl.cdiv(N + 1, 64 * NS) * (64 * NS)

    degs = _sc_degrees(sd2d, n_pad)

    h_pad = _tc_normalize(feats, degs[0], n_pad, 512)

    parts = _sc_gather_scatter(h_pad, sd2d, n_pad)

    return _tc_finalize(parts, degs[1], W, b.reshape(1, -1), N, 1024)


# speedup vs baseline: 11.0530x; 1.0420x over previous
import functools

import jax
import jax.numpy as jnp
from jax import lax
from jax.experimental import pallas as pl
from jax.experimental.pallas import tpu as pltpu
from jax.experimental.pallas import tpu_sc as plsc

NC = 2
NS = 16
LANES = 16
NW = NC * NS


def _sc_degrees(sd2d, n_pad):
    R, width = sd2d.shape[1], sd2d.shape[2]
    rpt = R // NS
    gpr = width // 16
    mesh = plsc.VectorSubcoreMesh(core_axis_name="c", subcore_axis_name="s", num_cores=NC, num_subcores=NS)

    @functools.partial(
        pl.kernel,
        out_type=jax.ShapeDtypeStruct((2, NS, n_pad), jnp.float32),
        mesh=mesh,
        scratch_types=[
            pltpu.VMEM((rpt, width), jnp.int32),
            pltpu.VMEM((n_pad,), jnp.float32),
        ],
        compiler_params=pltpu.CompilerParams(needs_layout_passes=False),
    )
    def deg_kernel(sd_hbm, out_hbm, idx_v, hist_v):
        c = lax.axis_index("c")
        s = lax.axis_index("s")
        pltpu.sync_copy(sd_hbm.at[c, pl.ds(s * rpt, rpt)], idx_v)
        z16 = jnp.zeros((16,), jnp.float32)

        def zbody(i, carry):
            hist_v[pl.ds(i * 16, 16)] = z16
            return carry

        lax.fori_loop(0, n_pad // 16, zbody, 0)
        ones16 = jnp.ones((16,), jnp.float32)

        def body(r, carry):
            for g in range(gpr):
                vec = idx_v[r, pl.ds(g * 16, 16)]
                plsc.addupdate_scatter(hist_v, [vec], ones16)
            return carry

        lax.fori_loop(0, rpt, body, 0)
        pltpu.sync_copy(hist_v, out_hbm.at[c, s])

    return deg_kernel(sd2d)


def _sc_gather_scatter(h_pad, sd2d, n_pad):
    D = h_pad.shape[1]
    R, width = sd2d.shape[1], sd2d.shape[2]
    rpt = R // NW
    zrows = n_pad // NS
    mesh = plsc.VectorSubcoreMesh(core_axis_name="c", subcore_axis_name="s", num_cores=NC, num_subcores=NS)

    nslot = 5
    gdep = 3
    nchunk = 5
    rptc = rpt // nchunk
    ntail = rptc % nslot
    assert rpt % nchunk == 0 and rptc % 8 == 0

    @functools.partial(
        pl.kernel,
        out_type=jax.ShapeDtypeStruct((2, n_pad, D), jnp.float32),
        mesh=mesh,
        scratch_types=[
            pltpu.VMEM((2, rptc, width), jnp.int32),
            pltpu.VMEM((width, D), jnp.float32),
            pltpu.VMEM((width, D), jnp.float32),
            pltpu.VMEM((width, D), jnp.float32),
            pltpu.VMEM((width, D), jnp.float32),
            pltpu.VMEM((width, D), jnp.float32),
            pltpu.SemaphoreType.DMA,
            pltpu.SemaphoreType.DMA,
            pltpu.SemaphoreType.DMA,
            pltpu.SemaphoreType.DMA,
            pltpu.SemaphoreType.DMA,
            pltpu.SemaphoreType.DMA,
            pltpu.SemaphoreType.DMA,
            pltpu.SemaphoreType.DMA,
            pltpu.SemaphoreType.DMA,
            pltpu.SemaphoreType.DMA,
            pltpu.VMEM_SHARED((n_pad, D), jnp.float32),
        ],
    )
    def gs_kernel(h_hbm, sd_hbm, out_hbm, idx_v,
                  b0, b1, b2, b3, b4, g0, g1, g2, g3, g4,
                  t0, t1, t2, t3, t4, agg_sh):
        bufs = (b0, b1, b2, b3, b4)
        gsem = (g0, g1, g2, g3, g4)
        ssem = (t0, t1, t2, t3, t4)
        c = lax.axis_index("c")
        s = lax.axis_index("s")
        w = c * NS + s
        z16 = jnp.zeros((16,), jnp.float32)

        def zbody(i, carry):
            b0[i // (D // 16), pl.ds((i % (D // 16)) * 16, 16)] = z16
            return carry

        lax.fori_loop(0, width * (D // 16), zbody, 0)
        for k in range(zrows // width):
            pltpu.sync_copy(b0, agg_sh.at[pl.ds(s * zrows + k * width, width)])
        zrem = zrows % width
        if zrem:
            pltpu.sync_copy(
                b0.at[pl.ds(0, zrem)],
                agg_sh.at[pl.ds(s * zrows + (zrows - zrem), zrem)])
        plsc.subcore_barrier()

        def outer(ch, carry):
            base = w * rpt + ch * rptc
            pltpu.sync_copy(sd_hbm.at[0, pl.ds(base, rptc)], idx_v.at[0])
            pltpu.sync_copy(sd_hbm.at[1, pl.ds(base, rptc)], idx_v.at[1])
            for b in range(gdep):
                pltpu.async_copy(h_hbm.at[idx_v.at[0, b]], bufs[b], gsem[b])

            def visit(j, b, nb, first, last):
                pltpu.make_async_copy(h_hbm.at[idx_v.at[0, j]], bufs[b],
                                      gsem[b]).wait()
                pltpu.async_copy(bufs[b], agg_sh.at[idx_v.at[1, j]],
                                 ssem[b], add=True)

                if not first:
                    pltpu.make_async_copy(
                        bufs[nb], agg_sh.at[idx_v.at[1, j]], ssem[nb]).wait()
                if not last:
                    pltpu.async_copy(h_hbm.at[idx_v.at[0, j + gdep]],
                                     bufs[nb], gsem[nb])

            def body(g, carry2):
                for u in range(nslot):
                    j = g * nslot + u

                    @pl.when(j >= nslot - gdep)
                    def _():
                        pltpu.make_async_copy(
                            bufs[(u + gdep) % nslot],
                            agg_sh.at[idx_v.at[1, j]],
                            ssem[(u + gdep) % nslot]).wait()

                    pltpu.make_async_copy(h_hbm.at[idx_v.at[0, j]], bufs[u],
                                          gsem[u]).wait()
                    pltpu.async_copy(bufs[u], agg_sh.at[idx_v.at[1, j]],
                                     ssem[u], add=True)

                    @pl.when(j + gdep < rptc)
                    def _():
                        pltpu.async_copy(h_hbm.at[idx_v.at[0, j + gdep]],
                                         bufs[(u + gdep) % nslot],
                                         gsem[(u + gdep) % nslot])
                return carry2

            lax.fori_loop(0, rptc // nslot, body, 0)
            for j in range(rptc - ntail, rptc):
                b = j % nslot
                nb = (j + gdep) % nslot
                pltpu.make_async_copy(
                    bufs[nb], agg_sh.at[idx_v.at[1, j]], ssem[nb]).wait()
                pltpu.make_async_copy(h_hbm.at[idx_v.at[0, j]], bufs[b],
                                      gsem[b]).wait()
                pltpu.async_copy(bufs[b], agg_sh.at[idx_v.at[1, j]],
                                 ssem[b], add=True)
            for j in range(rptc - (nslot - gdep), rptc):
                b = j % nslot
                pltpu.make_async_copy(bufs[b], agg_sh.at[idx_v.at[1, j]],
                                      ssem[b]).wait()
            return carry

        lax.fori_loop(0, nchunk, outer, 0)
        plsc.subcore_barrier()
        pltpu.sync_copy(agg_sh.at[pl.ds(s * zrows, zrows)],
                        out_hbm.at[c, pl.ds(s * zrows, zrows)])

    return gs_kernel(h_pad, sd2d)


def _tc_normalize(feats, deg_src, n_pad, blk):
    N, D = feats.shape

    def nk(f_ref, d_ref, o_ref):
        deg = jnp.sum(d_ref[...], axis=0)
        scale = lax.rsqrt(jnp.maximum(deg, 1.0)).reshape(blk, 1)
        o_ref[...] = f_ref[...] * scale

    return pl.pallas_call(
        nk,
        grid=(pl.cdiv(n_pad, blk),),
        in_specs=[
            pl.BlockSpec((blk, D), lambda i: (i, 0)),
            pl.BlockSpec((NS, blk), lambda i: (0, i)),
        ],
        out_specs=pl.BlockSpec((blk, D), lambda i: (i, 0)),
        out_shape=jax.ShapeDtypeStruct((n_pad, D), jnp.float32),
    )(feats, deg_src)


def _tc_finalize(parts, deg_dst, W, b2d, n_out, blk):
    D = parts.shape[2]
    DO = W.shape[1]

    def fk(p_ref, d_ref, w_ref, b_ref, o_ref):
        agg = p_ref[0] + p_ref[1]
        deg = jnp.sum(d_ref[...], axis=0)
        scale = lax.rsqrt(jnp.maximum(deg, 1.0)).reshape(d_ref.shape[1], 1)
        acc = jnp.dot(agg * scale, w_ref[...],
                      preferred_element_type=jnp.float32) + b_ref[...]
        o_ref[...] = jnp.maximum(acc, 0.0)

    return pl.pallas_call(
        fk,
        grid=(pl.cdiv(n_out, blk),),
        in_specs=[
            pl.BlockSpec((2, blk, D), lambda i: (0, i, 0)),
            pl.BlockSpec((NS, blk), lambda i: (0, i)),
            pl.BlockSpec((D, DO), lambda i: (0, 0)),
            pl.BlockSpec((1, DO), lambda i: (0, 0)),
        ],
        out_specs=pl.BlockSpec((blk, DO), lambda i: (i, 0)),
        out_shape=jax.ShapeDtypeStruct((n_out, DO), jnp.float32),
    )(parts, deg_dst, W, b2d)


def kernel(feats, edge_index, W, b):
    N, D = feats.shape
    E = edge_index.shape[1]

    width = 64
    epad = pl.cdiv(E, width * NW * 8) * (width * NW * 8)
    fill = N + (jnp.arange(epad - E, dtype=jnp.int32) % width)
    tail = jnp.broadcast_to(fill, (2, epad - E))
    sd = jnp.concatenate([edge_index.astype(jnp.int32), tail], axis=1)
    sd2d = sd.reshape(2, epad // width, width)

    n_pad = pl.cdiv(N + width, 128) * 128

    degs = _sc_degrees(sd2d, n_pad)

    h_pad = _tc_normalize(feats, degs[0], n_pad, 1280)

    parts = _sc_gather_scatter(h_pad, sd2d, n_pad)

    return _tc_finalize(parts, degs[1], W, b.reshape(1, -1), N, 1024)


# speedup vs baseline: 11.4058x; 1.0319x over previous
import functools

import jax
import jax.numpy as jnp
from jax import lax
from jax.experimental import pallas as pl
from jax.experimental.pallas import tpu as pltpu
from jax.experimental.pallas import tpu_sc as plsc

NC = 2
NS = 16
LANES = 16
NW = NC * NS


def _sc_degrees(arr2d, n_pad):
    R, width = arr2d.shape
    rpt = R // NW
    gpr = width // 16
    mesh = plsc.VectorSubcoreMesh(core_axis_name="c", subcore_axis_name="s", num_cores=NC, num_subcores=NS)

    @functools.partial(
        pl.kernel,
        out_type=jax.ShapeDtypeStruct((NW, n_pad), jnp.float32),
        mesh=mesh,
        scratch_types=[
            pltpu.VMEM((rpt, width), jnp.int32),
            pltpu.VMEM((n_pad,), jnp.float32),
        ],
        compiler_params=pltpu.CompilerParams(needs_layout_passes=False),
    )
    def deg_kernel(sd_hbm, out_hbm, idx_v, hist_v):
        c = lax.axis_index("c")
        s = lax.axis_index("s")
        w = c * NS + s
        pltpu.sync_copy(sd_hbm.at[pl.ds(w * rpt, rpt)], idx_v)
        z16 = jnp.zeros((16,), jnp.float32)

        def zbody(i, carry):
            hist_v[pl.ds(i * 16, 16)] = z16
            return carry

        lax.fori_loop(0, n_pad // 16, zbody, 0)
        ones16 = jnp.ones((16,), jnp.float32)

        def body(r, carry):
            for g in range(gpr):
                vec = idx_v[r, pl.ds(g * 16, 16)]
                plsc.addupdate_scatter(hist_v, [vec], ones16)
            return carry

        lax.fori_loop(0, rpt, body, 0)
        pltpu.sync_copy(hist_v, out_hbm.at[w])

    return deg_kernel(arr2d)


def _sc_gather_scatter(h_pad, sd2d, n_pad):
    D = h_pad.shape[1]
    R, width = sd2d.shape[1], sd2d.shape[2]
    rpt = R // NW
    zrows = n_pad // NS
    mesh = plsc.VectorSubcoreMesh(core_axis_name="c", subcore_axis_name="s", num_cores=NC, num_subcores=NS)

    nslot = 5
    gdep = 3
    nchunk = 5
    rptc = rpt // nchunk
    ntail = rptc % nslot
    assert rpt % nchunk == 0 and rptc % 8 == 0

    @functools.partial(
        pl.kernel,
        out_type=jax.ShapeDtypeStruct((2, n_pad, D), jnp.float32),
        mesh=mesh,
        scratch_types=[
            pltpu.VMEM((2, rptc, width), jnp.int32),
            pltpu.VMEM((width, D), jnp.float32),
            pltpu.VMEM((width, D), jnp.float32),
            pltpu.VMEM((width, D), jnp.float32),
            pltpu.VMEM((width, D), jnp.float32),
            pltpu.VMEM((width, D), jnp.float32),
            pltpu.SemaphoreType.DMA,
            pltpu.SemaphoreType.DMA,
            pltpu.SemaphoreType.DMA,
            pltpu.SemaphoreType.DMA,
            pltpu.SemaphoreType.DMA,
            pltpu.SemaphoreType.DMA,
            pltpu.SemaphoreType.DMA,
            pltpu.SemaphoreType.DMA,
            pltpu.SemaphoreType.DMA,
            pltpu.SemaphoreType.DMA,
            pltpu.VMEM_SHARED((n_pad, D), jnp.float32),
        ],
    )
    def gs_kernel(h_hbm, sd_hbm, out_hbm, idx_v,
                  b0, b1, b2, b3, b4, g0, g1, g2, g3, g4,
                  t0, t1, t2, t3, t4, agg_sh):
        bufs = (b0, b1, b2, b3, b4)
        gsem = (g0, g1, g2, g3, g4)
        ssem = (t0, t1, t2, t3, t4)
        c = lax.axis_index("c")
        s = lax.axis_index("s")
        w = c * NS + s
        z16 = jnp.zeros((16,), jnp.float32)

        def zbody(i, carry):
            b0[i // (D // 16), pl.ds((i % (D // 16)) * 16, 16)] = z16
            return carry

        lax.fori_loop(0, width * (D // 16), zbody, 0)
        for k in range(zrows // width):
            pltpu.sync_copy(b0, agg_sh.at[pl.ds(s * zrows + k * width, width)])
        zrem = zrows % width
        if zrem:
            pltpu.sync_copy(
                b0.at[pl.ds(0, zrem)],
                agg_sh.at[pl.ds(s * zrows + (zrows - zrem), zrem)])
        plsc.subcore_barrier()

        def outer(ch, carry):
            base = w * rpt + ch * rptc
            pltpu.sync_copy(sd_hbm.at[0, pl.ds(base, rptc)], idx_v.at[0])
            pltpu.sync_copy(sd_hbm.at[1, pl.ds(base, rptc)], idx_v.at[1])
            for b in range(gdep):
                pltpu.async_copy(h_hbm.at[idx_v.at[0, b]], bufs[b], gsem[b])

            def visit(j, b, nb, first, last):
                pltpu.make_async_copy(h_hbm.at[idx_v.at[0, j]], bufs[b],
                                      gsem[b]).wait()
                pltpu.async_copy(bufs[b], agg_sh.at[idx_v.at[1, j]],
                                 ssem[b], add=True)

                if not first:
                    pltpu.make_async_copy(
                        bufs[nb], agg_sh.at[idx_v.at[1, j]], ssem[nb]).wait()
                if not last:
                    pltpu.async_copy(h_hbm.at[idx_v.at[0, j + gdep]],
                                     bufs[nb], gsem[nb])

            def body(g, carry2):
                for u in range(nslot):
                    j = g * nslot + u

                    @pl.when(j >= nslot - gdep)
                    def _():
                        pltpu.make_async_copy(
                            bufs[(u + gdep) % nslot],
                            agg_sh.at[idx_v.at[1, j]],
                            ssem[(u + gdep) % nslot]).wait()

                    pltpu.make_async_copy(h_hbm.at[idx_v.at[0, j]], bufs[u],
                                          gsem[u]).wait()
                    pltpu.async_copy(bufs[u], agg_sh.at[idx_v.at[1, j]],
                                     ssem[u], add=True)

                    @pl.when(j + gdep < rptc)
                    def _():
                        pltpu.async_copy(h_hbm.at[idx_v.at[0, j + gdep]],
                                         bufs[(u + gdep) % nslot],
                                         gsem[(u + gdep) % nslot])
                return carry2

            lax.fori_loop(0, rptc // nslot, body, 0)
            for j in range(rptc - ntail, rptc):
                b = j % nslot
                nb = (j + gdep) % nslot
                pltpu.make_async_copy(
                    bufs[nb], agg_sh.at[idx_v.at[1, j]], ssem[nb]).wait()
                pltpu.make_async_copy(h_hbm.at[idx_v.at[0, j]], bufs[b],
                                      gsem[b]).wait()
                pltpu.async_copy(bufs[b], agg_sh.at[idx_v.at[1, j]],
                                 ssem[b], add=True)
            for j in range(rptc - (nslot - gdep), rptc):
                b = j % nslot
                pltpu.make_async_copy(bufs[b], agg_sh.at[idx_v.at[1, j]],
                                      ssem[b]).wait()
            return carry

        lax.fori_loop(0, nchunk, outer, 0)
        plsc.subcore_barrier()
        pltpu.sync_copy(agg_sh.at[pl.ds(s * zrows, zrows)],
                        out_hbm.at[c, pl.ds(s * zrows, zrows)])

    return gs_kernel(h_pad, sd2d)


def _tc_normalize(feats, deg_src, n_pad, blk):
    N, D = feats.shape

    def nk(f_ref, d_ref, o_ref):
        deg = jnp.sum(d_ref[...], axis=0)
        scale = lax.rsqrt(jnp.maximum(deg, 1.0)).reshape(blk, 1)
        o_ref[...] = f_ref[...] * scale

    return pl.pallas_call(
        nk,
        grid=(pl.cdiv(n_pad, blk),),
        in_specs=[
            pl.BlockSpec((blk, D), lambda i: (i, 0)),
            pl.BlockSpec((NW, blk), lambda i: (0, i)),
        ],
        out_specs=pl.BlockSpec((blk, D), lambda i: (i, 0)),
        out_shape=jax.ShapeDtypeStruct((n_pad, D), jnp.float32),
    )(feats, deg_src)


def _tc_finalize(parts, deg_dst, W, b2d, n_out, blk):
    D = parts.shape[2]
    DO = W.shape[1]

    def fk(p_ref, d_ref, w_ref, b_ref, o_ref):
        agg = p_ref[0] + p_ref[1]
        deg = jnp.sum(d_ref[...], axis=0)
        scale = lax.rsqrt(jnp.maximum(deg, 1.0)).reshape(d_ref.shape[1], 1)
        acc = jnp.dot(agg * scale, w_ref[...],
                      preferred_element_type=jnp.float32) + b_ref[...]
        o_ref[...] = jnp.maximum(acc, 0.0)

    return pl.pallas_call(
        fk,
        grid=(pl.cdiv(n_out, blk),),
        in_specs=[
            pl.BlockSpec((2, blk, D), lambda i: (0, i, 0)),
            pl.BlockSpec((NW, blk), lambda i: (0, i)),
            pl.BlockSpec((D, DO), lambda i: (0, 0)),
            pl.BlockSpec((1, DO), lambda i: (0, 0)),
        ],
        out_specs=pl.BlockSpec((blk, DO), lambda i: (i, 0)),
        out_shape=jax.ShapeDtypeStruct((n_out, DO), jnp.float32),
    )(parts, deg_dst, W, b2d)


def kernel(feats, edge_index, W, b):
    N, D = feats.shape
    E = edge_index.shape[1]

    width = 64
    epad = pl.cdiv(E, width * NW * 8) * (width * NW * 8)
    fill = N + (jnp.arange(epad - E, dtype=jnp.int32) % width)
    tail = jnp.broadcast_to(fill.reshape(1, -1, width),
                            (2, (epad - E) // width, width))
    ei3d = edge_index.astype(jnp.int32).reshape(2, E // width, width)
    sd2d = jnp.concatenate([ei3d, tail], axis=1)

    n_pad = pl.cdiv(N + width, 128) * 128

    deg_src = _sc_degrees(sd2d[0], n_pad)
    deg_dst = _sc_degrees(sd2d[1], n_pad)

    h_pad = _tc_normalize(feats, deg_src, n_pad, 1280)

    parts = _sc_gather_scatter(h_pad, sd2d, n_pad)

    return _tc_finalize(parts, deg_dst, W, b.reshape(1, -1), N, 1024)
